# baseline (speedup 1.0000x reference)
import functools

import jax
import jax.numpy as jnp
from jax import lax
from jax.experimental import pallas as pl
from jax.experimental.pallas import tpu as pltpu
from jax.experimental.pallas import tpu_sc as plsc

_NC = 2
_NS = 16
_NW = _NC * _NS
_L = 16

_CHUNK = 16384
_UNROLL = 8
_NACC = 2

_SC_FRAC_NUM, _SC_FRAC_DEN = 1, 2
_BR = 256


def _make_sc_count(n_sc):
    per_worker = n_sc // _NW
    n_chunks = per_worker // _CHUNK
    groups = _CHUNK // _L
    mesh = plsc.VectorSubcoreMesh(core_axis_name="c", subcore_axis_name="s")

    @functools.partial(
        pl.kernel,
        mesh=mesh,
        out_type=jax.ShapeDtypeStruct((_NW, 2, _L), jnp.int32),
        scratch_types=[
            pltpu.VMEM((2, _CHUNK), jnp.float32),
            pltpu.VMEM((2, _CHUNK), jnp.int32),
            pltpu.VMEM((2, _L), jnp.int32),
            pltpu.SemaphoreType.DMA,
            pltpu.SemaphoreType.DMA,
        ],
    )
    def sc_count(x_hbm, t_hbm, out_hbm, xbuf, tbuf, accbuf, sem0, sem1):
        wid = lax.axis_index("s") * _NC + lax.axis_index("c")
        base = wid * per_worker
        sems = (sem0, sem1)

        def copies(c, slot):
            off = base + c * _CHUNK
            return (
                pltpu.make_async_copy(
                    x_hbm.at[pl.ds(off, _CHUNK)], xbuf.at[slot], sems[slot]),
                pltpu.make_async_copy(
                    t_hbm.at[pl.ds(off, _CHUNK)], tbuf.at[slot], sems[slot]),
            )

        for cp in copies(0, 0):
            cp.start()

        zero = jnp.zeros((_L,), jnp.int32)
        acc = (zero,) * (2 * _NACC)
        for c in range(n_chunks):
            slot = c % 2
            if c + 1 < n_chunks:
                for cp in copies(c + 1, (c + 1) % 2):
                    cp.start()
            for cp in copies(c, slot):
                cp.wait()

            def group_body(g, gcarry, slot=slot):
                accs = list(gcarry)
                for u in range(_UNROLL):
                    off = g * (_L * _UNROLL) + u * _L
                    vx = xbuf[slot, pl.ds(off, _L)]
                    vt = tbuf[slot, pl.ds(off, _L)]
                    m = (vx < 0.5) & (vt == 0)
                    k = u % _NACC
                    accs[k] = accs[k] + jnp.where(m, 1, 0)
                    accs[_NACC + k] = accs[_NACC + k] + vt
                return tuple(accs)

            acc = lax.fori_loop(0, groups // _UNROLL, group_body, acc)

        accbuf[0, :] = functools.reduce(lambda a, b: a + b, acc[:_NACC])
        accbuf[1, :] = functools.reduce(lambda a, b: a + b, acc[_NACC:])
        pltpu.sync_copy(accbuf, out_hbm.at[wid])

    return sc_count


def _make_tc_count(rows):
    grid = rows // _BR

    def body(x_ref, t_ref, tn_ref, st_ref):
        @pl.when(pl.program_id(0) == 0)
        def _():
            tn_ref[...] = jnp.zeros_like(tn_ref)
            st_ref[...] = jnp.zeros_like(st_ref)

        x = x_ref[...]
        t = t_ref[...]
        m = (x < 0.5) & (t == 0)
        tn_ref[...] += jnp.sum(m.astype(jnp.float32), axis=0)
        st_ref[...] += jnp.sum(t.astype(jnp.float32), axis=0)

    return pl.pallas_call(
        body,
        grid=(grid,),
        in_specs=[
            pl.BlockSpec((_BR, 8, 128), lambda i: (i, 0, 0)),
            pl.BlockSpec((_BR, 8, 128), lambda i: (i, 0, 0)),
        ],
        out_specs=[
            pl.BlockSpec((8, 128), lambda i: (0, 0)),
            pl.BlockSpec((8, 128), lambda i: (0, 0)),
        ],
        out_shape=[
            jax.ShapeDtypeStruct((8, 128), jnp.float32),
            jax.ShapeDtypeStruct((8, 128), jnp.float32),
        ],
        compiler_params=pltpu.CompilerParams(
            dimension_semantics=("arbitrary",),
        ),
    )


def kernel(inputs, targets):
    n = inputs.shape[0]
    n_sc = (n * _SC_FRAC_NUM // _SC_FRAC_DEN) // (_NW * _CHUNK) * (_NW * _CHUNK)
    n_tc = n - n_sc
    rows = n_tc // 1024

    parts = _make_sc_count(n_sc)(inputs[:n_sc], targets[:n_sc])
    xr = inputs[n_sc:].reshape(rows, 8, 128)
    tr = targets[n_sc:].reshape(rows, 8, 128)
    tn_v, st_v = _make_tc_count(rows)(xr, tr)

    tn = parts[:, 0, :].sum().astype(jnp.float32) + tn_v.sum()
    st = parts[:, 1, :].sum().astype(jnp.float32) + st_v.sum()
    t0 = n - st
    return tn / jnp.clip(t0, 1e-12)


# speedup vs baseline: 1.7375x; 1.7375x over previous
import functools

import jax
import jax.numpy as jnp
from jax import lax
from jax.experimental import pallas as pl
from jax.experimental.pallas import tpu as pltpu
from jax.experimental.pallas import tpu_sc as plsc

_NC = 2
_NS = 16
_NW = _NC * _NS
_L = 16

_CHUNK = 16384
_UNROLL = 8
_NACC = 2

_SC_FRAC_NUM, _SC_FRAC_DEN = 1, 2
_BR = 256


def _make_sc_count(n_sc):
    per_worker = n_sc // _NW
    n_chunks = per_worker // _CHUNK
    groups = _CHUNK // _L
    mesh = plsc.VectorSubcoreMesh(core_axis_name="c", subcore_axis_name="s")

    @functools.partial(
        pl.kernel,
        mesh=mesh,
        out_type=jax.ShapeDtypeStruct((_NW, 2, _L), jnp.int32),
        scratch_types=[
            pltpu.VMEM((2, _CHUNK), jnp.float32),
            pltpu.VMEM((2, _CHUNK), jnp.int32),
            pltpu.VMEM((2, _L), jnp.int32),
            pltpu.SemaphoreType.DMA,
            pltpu.SemaphoreType.DMA,
        ],
    )
    def sc_count(x_hbm, t_hbm, out_hbm, xbuf, tbuf, accbuf, sem0, sem1):
        wid = lax.axis_index("s") * _NC + lax.axis_index("c")
        base = wid * per_worker
        sems = (sem0, sem1)

        def copies(c, slot):
            off = base + c * _CHUNK
            return (
                pltpu.make_async_copy(
                    x_hbm.at[pl.ds(off, _CHUNK)], xbuf.at[slot], sems[slot]),
                pltpu.make_async_copy(
                    t_hbm.at[pl.ds(off, _CHUNK)], tbuf.at[slot], sems[slot]),
            )

        for cp in copies(0, 0):
            cp.start()

        zero = jnp.zeros((_L,), jnp.int32)
        acc = (zero,) * (2 * _NACC)
        for c in range(n_chunks):
            slot = c % 2
            if c + 1 < n_chunks:
                for cp in copies(c + 1, (c + 1) % 2):
                    cp.start()
            for cp in copies(c, slot):
                cp.wait()

            def group_body(g, gcarry, slot=slot):
                accs = list(gcarry)
                for u in range(_UNROLL):
                    off = g * (_L * _UNROLL) + u * _L
                    vx = xbuf[slot, pl.ds(off, _L)]
                    vt = tbuf[slot, pl.ds(off, _L)]
                    m = (vx < 0.5) & (vt == 0)
                    k = u % _NACC
                    accs[k] = accs[k] + jnp.where(m, 1, 0)
                    accs[_NACC + k] = accs[_NACC + k] + vt
                return tuple(accs)

            acc = lax.fori_loop(0, groups // _UNROLL, group_body, acc)

        accbuf[0, :] = functools.reduce(lambda a, b: a + b, acc[:_NACC])
        accbuf[1, :] = functools.reduce(lambda a, b: a + b, acc[_NACC:])
        pltpu.sync_copy(accbuf, out_hbm.at[wid])

    return sc_count


def _make_tc_count(rows, row_off):
    grid = rows // _BR
    blk_off = row_off // _BR

    def body(x_ref, t_ref, tn_ref, st_ref):
        @pl.when(pl.program_id(0) == 0)
        def _():
            tn_ref[...] = jnp.zeros_like(tn_ref)
            st_ref[...] = jnp.zeros_like(st_ref)

        x = x_ref[...]
        t = t_ref[...]
        m = (x < 0.5) & (t == 0)
        tn_ref[...] += jnp.sum(m.astype(jnp.float32), axis=0)
        st_ref[...] += jnp.sum(t.astype(jnp.float32), axis=0)

    return pl.pallas_call(
        body,
        grid=(grid,),
        in_specs=[
            pl.BlockSpec((_BR, 8, 128), lambda i: (i + blk_off, 0, 0)),
            pl.BlockSpec((_BR, 8, 128), lambda i: (i + blk_off, 0, 0)),
        ],
        out_specs=[
            pl.BlockSpec((8, 128), lambda i: (0, 0)),
            pl.BlockSpec((8, 128), lambda i: (0, 0)),
        ],
        out_shape=[
            jax.ShapeDtypeStruct((8, 128), jnp.float32),
            jax.ShapeDtypeStruct((8, 128), jnp.float32),
        ],
        compiler_params=pltpu.CompilerParams(
            dimension_semantics=("arbitrary",),
        ),
    )


def kernel(inputs, targets):
    n = inputs.shape[0]
    n_sc = (n * _SC_FRAC_NUM // _SC_FRAC_DEN) // (_NW * _CHUNK) * (_NW * _CHUNK)
    n_tc = n - n_sc
    rows = n_tc // 1024

    parts = _make_sc_count(n_sc)(inputs, targets)
    xr = inputs.reshape(n // 1024, 8, 128)
    tr = targets.reshape(n // 1024, 8, 128)
    tn_v, st_v = _make_tc_count(rows, n_sc // 1024)(xr, tr)

    tn = parts[:, 0, :].sum().astype(jnp.float32) + tn_v.sum()
    st = parts[:, 1, :].sum().astype(jnp.float32) + st_v.sum()
    t0 = n - st
    return tn / jnp.clip(t0, 1e-12)
